# Initial kernel scaffold; baseline (speedup 1.0000x reference)
import jax
import jax.numpy as jnp
from jax.experimental import pallas as pl


def kernel(x, expert_ids, gmm1_weight, gmm1_weight_scale, gmm2_weight, gmm2_weight_scale, smooth_scales, expert_scales):
    raise NotImplementedError("write your pallas kernel here")


# speedup vs baseline: 1.2627x; 1.2627x over previous
import jax
import jax.numpy as jnp
from jax.experimental import pallas as pl

T = 128
K = 8
E = 64
D = 1024
F = 1024


def _moe_body(x_ref, ids_ref, w1_ref, s1_ref, w2_ref, s2_ref,
              smooth_ref, escale_ref, out_ref):
    e = pl.program_id(0)
    x = x_ref[...]
    h = jnp.dot(x, w1_ref[0], preferred_element_type=jnp.float32)
    h = h * s1_ref[0]
    gate = h[:, :F]
    up = h[:, F:]
    act = (gate * jax.nn.sigmoid(gate)) * up
    act = act * smooth_ref[0]
    y = jnp.dot(act, w2_ref[0], preferred_element_type=jnp.float32)
    y = y * s2_ref[0]
    coef = jnp.sum(
        jnp.where(ids_ref[...] == e, escale_ref[...], 0.0), axis=1)
    contrib = coef[:, None] * y

    @pl.when(e == 0)
    def _init():
        out_ref[...] = contrib

    @pl.when(e != 0)
    def _acc():
        out_ref[...] += contrib


def kernel(x, expert_ids, gmm1_weight, gmm1_weight_scale, gmm2_weight,
           gmm2_weight_scale, smooth_scales, expert_scales):
    return pl.pallas_call(
        _moe_body,
        grid=(E,),
        in_specs=[
            pl.BlockSpec((T, D), lambda e: (0, 0)),
            pl.BlockSpec((T, K), lambda e: (0, 0)),
            pl.BlockSpec((1, D, 2 * F), lambda e: (e, 0, 0)),
            pl.BlockSpec((1, 1, 2 * F), lambda e: (e, 0, 0)),
            pl.BlockSpec((1, F, D), lambda e: (e, 0, 0)),
            pl.BlockSpec((1, 1, D), lambda e: (e, 0, 0)),
            pl.BlockSpec((1, 1, F), lambda e: (e, 0, 0)),
            pl.BlockSpec((T, K), lambda e: (0, 0)),
        ],
        out_specs=pl.BlockSpec((T, D), lambda e: (0, 0)),
        out_shape=jax.ShapeDtypeStruct((T, D), jnp.float32),
    )(x, expert_ids, gmm1_weight, gmm1_weight_scale[:, None, :], gmm2_weight,
      gmm2_weight_scale[:, None, :], smooth_scales[:, None, :], expert_scales)
